# baseline (speedup 1.0000x reference)
import functools

import jax
import jax.numpy as jnp
from jax import lax
from jax.experimental import pallas as pl
from jax.experimental.pallas import tpu as pltpu
from jax.experimental.pallas import tpu_sc as plsc

NC = 2
NS = 16
NW = NC * NS
LANE = 128
SUB = 8


def _emb_lookup(b: int, h: int, d: int):
    bpw = b // NW
    nbc = bpw // LANE
    ndr = d // SUB
    ngrid = b // LANE
    assert bpw * NW == b and nbc * LANE == bpw and ndr * SUB == d

    @functools.partial(
        pl.kernel,
        mesh=plsc.VectorSubcoreMesh(core_axis_name="c", subcore_axis_name="s"),
        out_type=jax.ShapeDtypeStruct((h * ndr * ngrid * SUB * LANE,),
                                      jnp.float32),
        scratch_types=[
            pltpu.VMEM((2, bpw), jnp.int32),
            pltpu.VMEM((2, bpw, d), jnp.float32),
            pltpu.VMEM((2, ndr * nbc * SUB * LANE), jnp.float32),
            pltpu.SemaphoreType.DMA,
            pltpu.SemaphoreType.DMA,
            pltpu.SemaphoreType.DMA,
            pltpu.SemaphoreType.DMA,
        ],
        compiler_params=pltpu.CompilerParams(
            use_tc_tiling_on_sc=False, needs_layout_passes=False,
            disable_bounds_checks=True),
    )
    def body(idx_hbm, table_hbm, out_hbm, idx_v, rows_v, obuf,
             sem_i, sem_g, sem_w0, sem_w1):
        wid = lax.axis_index("s") * NC + lax.axis_index("c")
        b0 = wid * bpw
        bc0 = wid * nbc
        iota16 = jax.lax.iota(jnp.int32, 16)
        perm_lo = (iota16 // SUB) * (nbc * SUB * LANE) + (iota16 % SUB) * LANE
        perm_hi = perm_lo + 2 * (nbc * SUB * LANE)

        def stage(hh, slot):
            pltpu.async_copy(
                idx_hbm.at[pl.ds(hh * b + b0, bpw)], idx_v.at[slot], sem_i)

        def gather(slot):
            pltpu.async_copy(
                table_hbm.at[idx_v.at[slot]], rows_v.at[slot], sem_g)

        def wait_idx(slot):
            pltpu.make_async_copy(
                idx_hbm.at[pl.ds(b0, bpw)], idx_v.at[slot], sem_i).wait()

        def wait_gather(slot):
            pltpu.make_async_copy(
                table_hbm.at[idx_v.at[slot]], rows_v.at[slot], sem_g).wait()

        def fire_out(hh, slot, sem):
            for dr in range(ndr):
                pltpu.async_copy(
                    obuf.at[slot, pl.ds(dr * nbc * SUB * LANE, nbc * SUB * LANE)],
                    out_hbm.at[pl.ds(
                        ((hh * ndr + dr) * ngrid + bc0) * SUB * LANE,
                        nbc * SUB * LANE)],
                    sem)

        def wait_out(slot, sem):
            for dr in range(ndr):
                pltpu.make_async_copy(
                    obuf.at[slot, pl.ds(0, nbc * SUB * LANE)],
                    out_hbm.at[pl.ds(0, nbc * SUB * LANE)], sem).wait()

        stage(0, 0)
        wait_idx(0)
        gather(0)
        stage(1, 1)

        def h_body(hh, carry):
            slot = lax.rem(hh, 2)

            @pl.when(hh + 1 < h)
            def _():
                wait_idx(1 - slot)
            wait_gather(slot)

            @pl.when(hh + 1 < h)
            def _():
                gather(1 - slot)

            @pl.when(hh + 2 < h)
            def _():
                stage(hh + 2, slot)

            @pl.when((hh >= 2) & (slot == 0))
            def _():
                wait_out(0, sem_w0)

            @pl.when((hh >= 2) & (slot == 1))
            def _():
                wait_out(1, sem_w1)

            rows = rows_v.at[slot]
            odst = obuf.at[slot]

            def t_body(j, carry2):
                cj = (j // LANE) * (SUB * LANE) + lax.rem(j, LANE)
                v0 = rows[j, pl.ds(0, 16)]
                v1 = rows[j, pl.ds(16, 16)]
                plsc.store_scatter(odst, [perm_lo + cj], v0)
                plsc.store_scatter(odst, [perm_hi + cj], v1)
                return carry2

            lax.fori_loop(0, bpw, t_body, 0, unroll=8)

            @pl.when(slot == 0)
            def _():
                fire_out(hh, 0, sem_w0)

            @pl.when(slot == 1)
            def _():
                fire_out(hh, 1, sem_w1)

            return carry

        lax.fori_loop(0, h, h_body, 0)

        wait_out(0, sem_w0)
        wait_out(1, sem_w1)

    return body


def kernel(token_ids, weights):
    b, h = token_ids.shape
    d = weights.shape[1]
    flat_t = token_ids.T.reshape(-1).astype(jnp.int32)
    out1 = _emb_lookup(b, h, d)(flat_t, weights)
    out5 = out1.reshape(h, d // SUB, b // LANE, SUB, LANE)
    return out5.transpose(2, 4, 0, 1, 3).reshape(b, h, d)


# speedup vs baseline: 1.1133x; 1.1133x over previous
import functools

import jax
import jax.numpy as jnp
from jax import lax
from jax.experimental import pallas as pl
from jax.experimental.pallas import tpu as pltpu
from jax.experimental.pallas import tpu_sc as plsc

NC = 2
NS = 16
NW = NC * NS
LANE = 128
SUB = 8
G = 2


def _emb_lookup(b: int, h: int, d: int):
    bpw = b // NW
    nbc = bpw // LANE
    ndr = d // SUB
    ngrid = b // LANE
    hw = ndr * nbc * SUB * LANE
    nb = h // G
    assert bpw * NW == b and nbc * LANE == bpw and ndr * SUB == d
    assert nb * G == h

    @functools.partial(
        pl.kernel,
        mesh=plsc.VectorSubcoreMesh(core_axis_name="c", subcore_axis_name="s"),
        out_type=jax.ShapeDtypeStruct((h * ndr * ngrid * SUB * LANE,),
                                      jnp.float32),
        scratch_types=[
            pltpu.VMEM((2, G * bpw), jnp.int32),
            pltpu.VMEM((2, G * bpw, d), jnp.float32),
            pltpu.VMEM((2 * hw,), jnp.float32),
            pltpu.SemaphoreType.DMA,
            pltpu.SemaphoreType.DMA,
            pltpu.SemaphoreType.DMA,
        ],
        compiler_params=pltpu.CompilerParams(
            use_tc_tiling_on_sc=False, needs_layout_passes=False,
            disable_bounds_checks=True),
    )
    def body(idx_hbm, table_hbm, out_hbm, idx_v, rows_v, obuf,
             sem_i, sem_g, sem_w):
        wid = lax.axis_index("s") * NC + lax.axis_index("c")
        b0 = wid * bpw
        bc0 = wid * nbc
        iota16 = jax.lax.iota(jnp.int32, 16)
        perm_lo = (iota16 // SUB) * (nbc * SUB * LANE) + (iota16 % SUB) * LANE
        perm_hi = perm_lo + (16 // SUB) * (nbc * SUB * LANE)

        def stage(bi, slot):
            for g in range(G):
                pltpu.async_copy(
                    idx_hbm.at[pl.ds((bi * G + g) * b + b0, bpw)],
                    idx_v.at[slot, pl.ds(g * bpw, bpw)], sem_i)

        def wait_stage(slot):
            for g in range(G):
                pltpu.make_async_copy(
                    idx_hbm.at[pl.ds(b0, bpw)],
                    idx_v.at[slot, pl.ds(0, bpw)], sem_i).wait()

        def gather(slot):
            pltpu.async_copy(
                table_hbm.at[idx_v.at[slot]], rows_v.at[slot], sem_g)

        def wait_gather(slot):
            pltpu.make_async_copy(
                table_hbm.at[idx_v.at[slot]], rows_v.at[slot], sem_g).wait()

        def fire_out(bi):
            for g in range(G):
                for dr in range(ndr):
                    pltpu.async_copy(
                        obuf.at[pl.ds(g * hw + dr * nbc * SUB * LANE,
                                      nbc * SUB * LANE)],
                        out_hbm.at[pl.ds(
                            (((bi * G + g) * ndr + dr) * ngrid + bc0)
                            * SUB * LANE,
                            nbc * SUB * LANE)],
                        sem_w)

        def wait_out():
            for _ in range(G * ndr):
                pltpu.make_async_copy(
                    obuf.at[pl.ds(0, nbc * SUB * LANE)],
                    out_hbm.at[pl.ds(0, nbc * SUB * LANE)], sem_w).wait()

        stage(0, 0)
        wait_stage(0)
        gather(0)
        stage(1, 1)

        def batch(bi, carry):
            slot = lax.rem(bi, 2)

            @pl.when(bi + 1 < nb)
            def _():
                wait_stage(1 - slot)
            wait_gather(slot)

            @pl.when(bi + 1 < nb)
            def _():
                gather(1 - slot)

            @pl.when(bi + 2 < nb)
            def _():
                stage(bi + 2, slot)

            @pl.when(bi >= 1)
            def _():
                wait_out()

            rows = rows_v.at[slot]

            @plsc.parallel_loop(0, G * bpw, unroll=8)
            def _(j):
                hsub = j // bpw
                jj = lax.rem(j, bpw)
                cj = (hsub * hw + (jj // LANE) * (SUB * LANE)
                      + lax.rem(jj, LANE))
                plsc.store_scatter(obuf, [perm_lo + cj],
                                   rows[j, pl.ds(0, 16)])
                plsc.store_scatter(obuf, [perm_hi + cj],
                                   rows[j, pl.ds(16, 16)])

            fire_out(bi)
            return carry

        lax.fori_loop(0, nb, batch, 0)
        wait_out()

    return body


def kernel(token_ids, weights):
    b, h = token_ids.shape
    d = weights.shape[1]
    flat_t = token_ids.T.reshape(-1).astype(jnp.int32)
    out1 = _emb_lookup(b, h, d)(flat_t, weights)
    out5 = out1.reshape(h, d // SUB, b // LANE, SUB, LANE)
    return out5.transpose(2, 4, 0, 1, 3).reshape(b, h, d)
